# baseline (speedup 1.0000x reference)
import functools

import jax
import jax.numpy as jnp
from jax import lax
from jax.experimental import pallas as pl
from jax.experimental.pallas import tpu as pltpu
from jax.experimental.pallas import tpu_sc as plsc

B = 4
N = 307200
D = 16
NB_MATCH = 5000
NB_NONMATCH = 50000
NB_SAMPLE = NB_NONMATCH // NB_MATCH
ALPHA = 0.5
TOTAL = B * NB_NONMATCH
NW = 32
C = 80
CM = C // NB_SAMPLE
CPW = 80
ROWS_PER_W = C * CPW
PAD_TOTAL = NW * ROWS_PER_W
TBL = B * N
G = TBL // 8
MAXOFF = (B - 1) * N

NT = N // 128
TPT = NT // 8
BT = 10
NBLK = TPT // BT
BIN = BT * 8
BOUT = BT * 16

_mesh = plsc.VectorSubcoreMesh(core_axis_name="c", subcore_axis_name="s")
_params = pltpu.CompilerParams(needs_layout_passes=False)


@functools.partial(
    pl.kernel,
    mesh=_mesh,
    compiler_params=_params,
    out_type=(jax.ShapeDtypeStruct((G, 128), jnp.float32),
              jax.ShapeDtypeStruct((G, 128), jnp.float32)),
    scratch_types=[
        pltpu.VMEM((2 * BIN, 128), jnp.float32),
        pltpu.VMEM((2 * BIN, 128), jnp.float32),
        pltpu.VMEM((BOUT, 128), jnp.float32),
        pltpu.VMEM((BOUT, 128), jnp.float32),
        pltpu.SemaphoreType.DMA,
        pltpu.SemaphoreType.DMA,
        pltpu.SemaphoreType.DMA,
        pltpu.SemaphoreType.DMA,
    ],
)
def _format_sc(vA, vB, tA, tB, in0, in1, ou0, ou1, is0, is1, os0, os1):
    wid = lax.axis_index("s") * 2 + lax.axis_index("c")
    b = wid // 8
    t0 = (wid % 8) * TPT
    iota = lax.iota(jnp.int32, 16)
    rvec = (iota >> 3) * BIN + (iota & 7)

    for vin, vout in ((vA, tA), (vB, tB)):

        def fire_in(blk, ibuf, sem):
            tb = t0 + blk * BT
            cps = []
            for r in range(2):
                rs = pl.multiple_of((b * 4800 + r * 2400 + tb) * 8, 8)
                cps.append(pltpu.async_copy(
                    vin.at[pl.ds(rs, BIN)],
                    ibuf.at[pl.ds(r * BIN, BIN)], sem))
            return cps

        def fire_out(blk, obuf, sem):
            gs = pl.multiple_of(b * (N // 8) + (t0 + blk * BT) * 16, 8)
            return pltpu.async_copy(obuf, vout.at[pl.ds(gs, BOUT)], sem)

        def transform(ibuf, obuf):
            def kbody(k, carry):
                t = k // 16
                kp = k % 16
                rv = rvec + t * 8
                for j in range(8):
                    cv = jnp.broadcast_to(kp * 8 + j, (16,))
                    obuf[k, pl.ds(j * 16, 16)] = plsc.load_gather(
                        ibuf, [rv, cv])
                return carry

            lax.fori_loop(0, BOUT, kbody, 0)

        def blk_body(i, carry):
            b0i = 2 * i
            b1i = 2 * i + 1
            cin0 = fire_in(b0i, in0, is0)
            cin1 = fire_in(b1i, in1, is1)
            for cp in cin0:
                cp.wait()
            transform(in0, ou0)
            cout0 = fire_out(b0i, ou0, os0)
            for cp in cin1:
                cp.wait()
            transform(in1, ou1)
            cout1 = fire_out(b1i, ou1, os1)
            cout0.wait()
            cout1.wait()
            return carry

        lax.fori_loop(0, NBLK // 2, blk_body, 0)


@functools.partial(
    pl.kernel,
    mesh=_mesh,
    compiler_params=_params,
    out_type=jax.ShapeDtypeStruct((NW, 16), jnp.float32),
    scratch_types=[
        pltpu.VMEM((CPW, C), jnp.int32),
        pltpu.VMEM((ROWS_PER_W,), jnp.int32),
        pltpu.VMEM((CPW, C), jnp.int32),
        pltpu.VMEM((ROWS_PER_W,), jnp.int32),
        pltpu.VMEM((CPW // 2, 16), jnp.int32),
        pltpu.VMEM((ROWS_PER_W // NB_SAMPLE,), jnp.int32),
        pltpu.VMEM((C, 128), jnp.float32),
        pltpu.VMEM((C, 128), jnp.float32),
        pltpu.VMEM((C, 128), jnp.float32),
        pltpu.VMEM((C, 128), jnp.float32),
        pltpu.VMEM((C, 128), jnp.float32),
        pltpu.VMEM((C, 128), jnp.float32),
        pltpu.VMEM((C, 128), jnp.float32),
        pltpu.VMEM((C, 128), jnp.float32),
        pltpu.VMEM((16, 128), jnp.float32),
        pltpu.VMEM((16, 128), jnp.float32),
        pltpu.VMEM((16,), jnp.float32),
        pltpu.SemaphoreType.DMA,
        pltpu.SemaphoreType.DMA,
        pltpu.SemaphoreType.DMA,
        pltpu.SemaphoreType.DMA,
        pltpu.SemaphoreType.DMA,
        pltpu.SemaphoreType.DMA,
    ],
)
def _triplet_sc(tblA, tblB, idxA_h, mB_h, idxBn_h, out_h,
                gA, wA, gBn, wBn, gBp, wBp,
                rA0, rA1, rA2, rA3, rBn0, rBn1, rBn2, rBn3, rBp0, rBp1,
                accv, sm0, sm1, sm2, sm3, bs0, bs1):
    wid = lax.axis_index("s") * 2 + lax.axis_index("c")
    iota = lax.iota(jnp.int32, 16)
    w0 = pl.multiple_of(wid * ROWS_PER_W, ROWS_PER_W)
    m0 = pl.multiple_of(wid * (ROWS_PER_W // NB_SAMPLE), 8)

    pltpu.sync_copy(idxA_h.at[pl.ds(wid * CPW, CPW)], gA)
    pltpu.sync_copy(idxBn_h.at[pl.ds(wid * CPW, CPW)], gBn)
    pltpu.sync_copy(mB_h.at[pl.ds(wid * (CPW // 2), CPW // 2)], gBp)

    VPR = C // 16

    def adj_body(j, carry):
        r = j // VPR
        co = (j % VPR) * 16
        sl = pl.ds(j * 16, 16)
        base = w0 + j * 16
        b0 = base // NB_NONMATCH
        off = jnp.broadcast_to(jnp.minimum(b0, B - 1) * N, (16,))
        va = gA[r, pl.ds(co, 16)] + off
        wA[sl] = (va & 7) << 4
        gA[r, pl.ds(co, 16)] = va >> 3
        vn = gBn[r, pl.ds(co, 16)] + off
        wBn[sl] = (vn & 7) << 4
        gBn[r, pl.ds(co, 16)] = vn >> 3
        return carry

    lax.fori_loop(0, ROWS_PER_W // 16, adj_body, 0)

    def adjm_body(j, carry):
        sl = pl.ds(j * 16, 16)
        basem = m0 + j * 16
        b0 = basem // NB_MATCH
        r0 = basem % NB_MATCH
        bump = jnp.where(iota >= NB_MATCH - r0, N, 0)
        off = jnp.minimum(jnp.minimum(b0, B - 1) * N + bump, MAXOFF)
        vp = gBp[j, :] + off
        wBp[sl] = (vp & 7) << 4
        gBp[j, :] = vp >> 3
        return carry

    lax.fori_loop(0, ROWS_PER_W // NB_SAMPLE // 16, adjm_body, 0)

    def compute(c, bA, bBn, bBp, ub, acc):
        def g_body(g, a):
            t0 = g * 16
            rvec = t0 + iota
            u0 = t0 // NB_SAMPLE
            ur = t0 % NB_SAMPLE
            uloc = (u0
                    + jnp.where(iota >= NB_SAMPLE - ur, 1, 0)
                    + jnp.where(iota >= 2 * NB_SAMPLE - ur, 1, 0))
            wAv = wA[pl.ds(c * C + t0, 16)]
            wBnv = wBn[pl.ds(c * C + t0, 16)]
            wBpv = plsc.load_gather(wBp, [c * CM + uloc])
            uvec = uloc + ub
            sacc = jnp.zeros((16,), jnp.float32)
            for d in range(D):
                aa = plsc.load_gather(bA, [rvec, wAv + d])
                bn = plsc.load_gather(bBn, [rvec, wBnv + d])
                bp = plsc.load_gather(bBp, [uvec, wBpv + d])
                sacc = sacc + (bn - bp) * (aa + aa - bp - bn)
            return a + jnp.maximum(sacc + ALPHA, 0.0)

        return lax.fori_loop(0, C // 16, g_body, acc)

    rAs = (rA0, rA1, rA2, rA3)
    rBns = (rBn0, rBn1, rBn2, rBn3)
    sms = (sm0, sm1, sm2, sm3)
    rBps = (rBp0, rBp1)
    bss = (bs0, bs1)

    def chunk_body(i, acc):
        c0 = 4 * i
        cps = []
        for k in range(4):
            cps.append(pltpu.async_copy(
                tblA.at[gA.at[c0 + k]], rAs[k], sms[k]))
            cps.append(pltpu.async_copy(
                tblB.at[gBn.at[c0 + k]], rBns[k], sms[k]))
        bcps = []
        for p in range(2):
            bcps.append(pltpu.async_copy(
                tblB.at[gBp.at[2 * i + p]], rBps[p], bss[p]))
        for k in range(4):
            p = k // 2
            if k % 2 == 0:
                bcps[p].wait()
            cps[2 * k].wait()
            cps[2 * k + 1].wait()
            acc = compute(c0 + k, rAs[k], rBns[k], rBps[p],
                          (k % 2) * CM, acc)
        return acc

    acc = lax.fori_loop(0, CPW // 4, chunk_body, jnp.zeros((16,), jnp.float32))
    accv[...] = acc
    pltpu.sync_copy(accv, out_h.at[wid])


def _native_view(x):
    w = x.squeeze(1).transpose(0, 2, 1)
    v = w.reshape(B, 2, 8, NT, 128).transpose(0, 1, 3, 2, 4)
    return v.reshape(B * 2 * NT * 8, 128)


def kernel(outA, outB, matchA, matchB, nonMatchA, nonMatchB, device):
    tblA, tblB = _format_sc(_native_view(outA), _native_view(outB))
    idxA = jnp.pad(nonMatchA.reshape(-1),
                   (0, PAD_TOTAL - TOTAL)).reshape(NW * CPW, C)
    idxBn = jnp.pad(nonMatchB.reshape(-1),
                    (0, PAD_TOTAL - TOTAL)).reshape(NW * CPW, C)
    mB = jnp.pad(matchB.reshape(-1),
                 (0, (PAD_TOTAL - TOTAL) // NB_SAMPLE)).reshape(
                     NW * (CPW // 2), 2 * CM)
    partials = _triplet_sc(tblA, tblB, idxA, mB, idxBn)
    pad_fix = 0.5 * (PAD_TOTAL - TOTAL)
    return (jnp.sum(partials) - pad_fix) / NB_MATCH


# speedup vs baseline: 1.0471x; 1.0471x over previous
import functools

import jax
import jax.numpy as jnp
from jax import lax
from jax.experimental import pallas as pl
from jax.experimental.pallas import tpu as pltpu
from jax.experimental.pallas import tpu_sc as plsc

B = 4
N = 307200
D = 16
NB_MATCH = 5000
NB_NONMATCH = 50000
NB_SAMPLE = NB_NONMATCH // NB_MATCH
ALPHA = 0.5
TOTAL = B * NB_NONMATCH
NW = 32
C = 80
CM = C // NB_SAMPLE
CPW = 80
ROWS_PER_W = C * CPW
PAD_TOTAL = NW * ROWS_PER_W
TBL = B * N
G = TBL // 8
MAXOFF = (B - 1) * N

NT = N // 128
TPT = NT // 8
BT = 10
NBLK = TPT // BT
BIN = BT * 8
BOUT = BT * 16

_mesh = plsc.VectorSubcoreMesh(core_axis_name="c", subcore_axis_name="s")
_params = pltpu.CompilerParams(needs_layout_passes=False)


@functools.partial(
    pl.kernel,
    mesh=_mesh,
    compiler_params=_params,
    out_type=(jax.ShapeDtypeStruct((G, 128), jnp.float32),
              jax.ShapeDtypeStruct((G, 128), jnp.float32)),
    scratch_types=[
        pltpu.VMEM((2 * BIN, 128), jnp.float32),
        pltpu.VMEM((2 * BIN, 128), jnp.float32),
        pltpu.VMEM((BOUT, 128), jnp.float32),
        pltpu.VMEM((BOUT, 128), jnp.float32),
        pltpu.SemaphoreType.DMA,
        pltpu.SemaphoreType.DMA,
        pltpu.SemaphoreType.DMA,
        pltpu.SemaphoreType.DMA,
    ],
)
def _format_sc(vA, vB, tA, tB, in0, in1, ou0, ou1, is0, is1, os0, os1):
    wid = lax.axis_index("s") * 2 + lax.axis_index("c")
    b = wid // 8
    t0 = (wid % 8) * TPT
    iota = lax.iota(jnp.int32, 16)
    rvec = (iota >> 3) * BIN + (iota & 7)

    for vin, vout in ((vA, tA), (vB, tB)):

        def fire_in(blk, ibuf, sem):
            tb = t0 + blk * BT
            cps = []
            for r in range(2):
                rs = pl.multiple_of((b * 4800 + r * 2400 + tb) * 8, 8)
                cps.append(pltpu.async_copy(
                    vin.at[pl.ds(rs, BIN)],
                    ibuf.at[pl.ds(r * BIN, BIN)], sem))
            return cps

        def fire_out(blk, obuf, sem):
            gs = pl.multiple_of(b * (N // 8) + (t0 + blk * BT) * 16, 8)
            return pltpu.async_copy(obuf, vout.at[pl.ds(gs, BOUT)], sem)

        def transform(ibuf, obuf):
            def kbody(k, carry):
                t = k // 16
                kp = k % 16
                rv = rvec + t * 8
                for j in range(8):
                    cv = jnp.broadcast_to(kp * 8 + j, (16,))
                    obuf[k, pl.ds(j * 16, 16)] = plsc.load_gather(
                        ibuf, [rv, cv])
                return carry

            lax.fori_loop(0, BOUT, kbody, 0)

        def blk_body(i, carry):
            b0i = 2 * i
            b1i = 2 * i + 1
            cin0 = fire_in(b0i, in0, is0)
            cin1 = fire_in(b1i, in1, is1)
            for cp in cin0:
                cp.wait()
            transform(in0, ou0)
            cout0 = fire_out(b0i, ou0, os0)
            for cp in cin1:
                cp.wait()
            transform(in1, ou1)
            cout1 = fire_out(b1i, ou1, os1)
            cout0.wait()
            cout1.wait()
            return carry

        lax.fori_loop(0, NBLK // 2, blk_body, 0)


@functools.partial(
    pl.kernel,
    mesh=_mesh,
    compiler_params=_params,
    out_type=jax.ShapeDtypeStruct((NW, 16), jnp.float32),
    scratch_types=[
        pltpu.VMEM((CPW, C), jnp.int32),
        pltpu.VMEM((ROWS_PER_W,), jnp.int32),
        pltpu.VMEM((CPW, C), jnp.int32),
        pltpu.VMEM((ROWS_PER_W,), jnp.int32),
        pltpu.VMEM((CPW // 2, 16), jnp.int32),
        pltpu.VMEM((ROWS_PER_W // NB_SAMPLE,), jnp.int32),
        pltpu.VMEM((C, 128), jnp.float32),
        pltpu.VMEM((C, 128), jnp.float32),
        pltpu.VMEM((C, 128), jnp.float32),
        pltpu.VMEM((C, 128), jnp.float32),
        pltpu.VMEM((C, 128), jnp.float32),
        pltpu.VMEM((C, 128), jnp.float32),
        pltpu.VMEM((C, 128), jnp.float32),
        pltpu.VMEM((C, 128), jnp.float32),
        pltpu.VMEM((16, 128), jnp.float32),
        pltpu.VMEM((16, 128), jnp.float32),
        pltpu.VMEM((16,), jnp.float32),
        pltpu.SemaphoreType.DMA,
        pltpu.SemaphoreType.DMA,
        pltpu.SemaphoreType.DMA,
        pltpu.SemaphoreType.DMA,
        pltpu.SemaphoreType.DMA,
        pltpu.SemaphoreType.DMA,
    ],
)
def _triplet_sc(tblA, tblB, idxA_h, mB_h, idxBn_h, out_h,
                gA, wA, gBn, wBn, gBp, wBp,
                rA0, rA1, rA2, rA3, rBn0, rBn1, rBn2, rBn3, rBp0, rBp1,
                accv, sm0, sm1, sm2, sm3, bs0, bs1):
    wid = lax.axis_index("s") * 2 + lax.axis_index("c")
    iota = lax.iota(jnp.int32, 16)
    w0 = pl.multiple_of(wid * ROWS_PER_W, ROWS_PER_W)
    m0 = pl.multiple_of(wid * (ROWS_PER_W // NB_SAMPLE), 8)

    pltpu.sync_copy(idxA_h.at[pl.ds(wid * CPW, CPW)], gA)
    pltpu.sync_copy(idxBn_h.at[pl.ds(wid * CPW, CPW)], gBn)
    pltpu.sync_copy(mB_h.at[pl.ds(wid * (CPW // 2), CPW // 2)], gBp)

    VPR = C // 16

    def adj_body(j, carry):
        r = j // VPR
        co = (j % VPR) * 16
        sl = pl.ds(j * 16, 16)
        base = w0 + j * 16
        b0 = base // NB_NONMATCH
        off = jnp.broadcast_to(jnp.minimum(b0, B - 1) * N, (16,))
        va = gA[r, pl.ds(co, 16)] + off
        wA[sl] = (va & 7) << 4
        gA[r, pl.ds(co, 16)] = va >> 3
        vn = gBn[r, pl.ds(co, 16)] + off
        wBn[sl] = (vn & 7) << 4
        gBn[r, pl.ds(co, 16)] = vn >> 3
        return carry

    lax.fori_loop(0, ROWS_PER_W // 16, adj_body, 0)

    def adjm_body(j, carry):
        sl = pl.ds(j * 16, 16)
        basem = m0 + j * 16
        b0 = basem // NB_MATCH
        r0 = basem % NB_MATCH
        bump = jnp.where(iota >= NB_MATCH - r0, N, 0)
        off = jnp.minimum(jnp.minimum(b0, B - 1) * N + bump, MAXOFF)
        vp = gBp[j, :] + off
        wBp[sl] = (vp & 7) << 4
        gBp[j, :] = vp >> 3
        return carry

    lax.fori_loop(0, ROWS_PER_W // NB_SAMPLE // 16, adjm_body, 0)

    def compute(c, bA, bBn, bBp, ub, acc):
        def g_body(g, a):
            t0 = g * 16
            rvec = t0 + iota
            u0 = t0 // NB_SAMPLE
            ur = t0 % NB_SAMPLE
            uloc = (u0
                    + jnp.where(iota >= NB_SAMPLE - ur, 1, 0)
                    + jnp.where(iota >= 2 * NB_SAMPLE - ur, 1, 0))
            wAv = wA[pl.ds(c * C + t0, 16)]
            wBnv = wBn[pl.ds(c * C + t0, 16)]
            wBpv = plsc.load_gather(wBp, [c * CM + uloc])
            uvec = uloc + ub
            sacc = jnp.zeros((16,), jnp.float32)
            for dd in range(D):
                dv = (iota + dd) & 15
                aa = plsc.load_gather(bA, [rvec, wAv + dv])
                bn = plsc.load_gather(bBn, [rvec, wBnv + dv])
                bp = plsc.load_gather(bBp, [uvec, wBpv + dv])
                sacc = sacc + (bn - bp) * (aa + aa - bp - bn)
            return a + jnp.maximum(sacc + ALPHA, 0.0)

        return lax.fori_loop(0, C // 16, g_body, acc)

    rAs = (rA0, rA1, rA2, rA3)
    rBns = (rBn0, rBn1, rBn2, rBn3)
    sms = (sm0, sm1, sm2, sm3)
    rBps = (rBp0, rBp1)
    bss = (bs0, bs1)

    def chunk_body(i, acc):
        c0 = 4 * i
        cps = []
        for k in range(4):
            cps.append(pltpu.async_copy(
                tblA.at[gA.at[c0 + k]], rAs[k], sms[k]))
            cps.append(pltpu.async_copy(
                tblB.at[gBn.at[c0 + k]], rBns[k], sms[k]))
        bcps = []
        for p in range(2):
            bcps.append(pltpu.async_copy(
                tblB.at[gBp.at[2 * i + p]], rBps[p], bss[p]))
        for k in range(4):
            p = k // 2
            if k % 2 == 0:
                bcps[p].wait()
            cps[2 * k].wait()
            cps[2 * k + 1].wait()
            acc = compute(c0 + k, rAs[k], rBns[k], rBps[p],
                          (k % 2) * CM, acc)
        return acc

    acc = lax.fori_loop(0, CPW // 4, chunk_body, jnp.zeros((16,), jnp.float32))
    accv[...] = acc
    pltpu.sync_copy(accv, out_h.at[wid])


def _native_view(x):
    w = x.squeeze(1).transpose(0, 2, 1)
    v = w.reshape(B, 2, 8, NT, 128).transpose(0, 1, 3, 2, 4)
    return v.reshape(B * 2 * NT * 8, 128)


def kernel(outA, outB, matchA, matchB, nonMatchA, nonMatchB, device):
    tblA, tblB = _format_sc(_native_view(outA), _native_view(outB))
    idxA = jnp.pad(nonMatchA.reshape(-1),
                   (0, PAD_TOTAL - TOTAL)).reshape(NW * CPW, C)
    idxBn = jnp.pad(nonMatchB.reshape(-1),
                    (0, PAD_TOTAL - TOTAL)).reshape(NW * CPW, C)
    mB = jnp.pad(matchB.reshape(-1),
                 (0, (PAD_TOTAL - TOTAL) // NB_SAMPLE)).reshape(
                     NW * (CPW // 2), 2 * CM)
    partials = _triplet_sc(tblA, tblB, idxA, mB, idxBn)
    pad_fix = 0.5 * (PAD_TOTAL - TOTAL)
    return (jnp.sum(partials) - pad_fix) / NB_MATCH


# speedup vs baseline: 1.2422x; 1.1863x over previous
import functools

import jax
import jax.numpy as jnp
from jax import lax
from jax.experimental import pallas as pl
from jax.experimental.pallas import tpu as pltpu
from jax.experimental.pallas import tpu_sc as plsc

B = 4
N = 307200
D = 16
NB_MATCH = 5000
NB_NONMATCH = 50000
NB_SAMPLE = NB_NONMATCH // NB_MATCH
ALPHA = 0.5
TOTAL = B * NB_NONMATCH
NW = 32
C = 320
CM = C // NB_SAMPLE
CPW = 20
ROWS_PER_W = C * CPW
PAD_TOTAL = NW * ROWS_PER_W
TBL = B * N
G = TBL // 8
MAXOFF = (B - 1) * N

_mesh = plsc.VectorSubcoreMesh(core_axis_name="c", subcore_axis_name="s")


@functools.partial(
    pl.kernel,
    mesh=_mesh,
    compiler_params=pltpu.CompilerParams(needs_layout_passes=False),
    out_type=jax.ShapeDtypeStruct((NW, 16), jnp.float32),
    scratch_types=[
        pltpu.VMEM((C,), jnp.int32),
        pltpu.VMEM((C,), jnp.int32),
        pltpu.VMEM((C,), jnp.int32),
        pltpu.VMEM((C,), jnp.int32),
        pltpu.VMEM((CM,), jnp.int32),
        pltpu.VMEM((CM,), jnp.int32),
        pltpu.VMEM((C, 128), jnp.float32),
        pltpu.VMEM((C, 128), jnp.float32),
        pltpu.VMEM((CM, 128), jnp.float32),
        pltpu.VMEM((16,), jnp.float32),
        pltpu.SemaphoreType.DMA,
    ],
)
def _triplet_sc(tblA, tblB, idxA_h, mB_h, idxBn_h, out_h,
                gA_v, wA_v, gBn_v, wBn_v, gBp_v, wBp_v,
                rA, rBn, rBp, accv, sem):
    wid = lax.axis_index("s") * 2 + lax.axis_index("c")

    def chunk_body(c, acc):
        s = pl.multiple_of(wid * ROWS_PER_W + c * C, C)
        sm = pl.multiple_of(s // NB_SAMPLE, CM)
        pltpu.sync_copy(idxA_h.at[pl.ds(s, C)], gA_v)
        pltpu.sync_copy(idxBn_h.at[pl.ds(s, C)], gBn_v)
        pltpu.sync_copy(mB_h.at[pl.ds(sm, CM)], gBp_v)

        def adj_body(j, carry):
            sl = pl.ds(j * 16, 16)
            iota = lax.iota(jnp.int32, 16)
            base = s + j * 16
            b0 = base // NB_NONMATCH
            r0 = base % NB_NONMATCH
            bump = jnp.where(iota >= NB_NONMATCH - r0, N, 0)
            off = jnp.minimum(b0 * N + bump, MAXOFF)
            va = gA_v[sl] + off
            wA_v[sl] = (va & 7) << 4
            gA_v[sl] = va >> 3
            vn = gBn_v[sl] + off
            wBn_v[sl] = (vn & 7) << 4
            gBn_v[sl] = vn >> 3
            return carry

        lax.fori_loop(0, C // 16, adj_body, 0)

        for j in range(CM // 16):
            sl = pl.ds(j * 16, 16)
            iota = lax.iota(jnp.int32, 16)
            basem = sm + j * 16
            b0 = basem // NB_MATCH
            r0 = basem % NB_MATCH
            bump = jnp.where(iota >= NB_MATCH - r0, N, 0)
            off = jnp.minimum(b0 * N + bump, MAXOFF)
            vp = gBp_v[sl] + off
            wBp_v[sl] = (vp & 7) << 4
            gBp_v[sl] = vp >> 3

        cpA = pltpu.async_copy(tblA.at[gA_v], rA, sem)
        cpBn = pltpu.async_copy(tblB.at[gBn_v], rBn, sem)
        cpBp = pltpu.async_copy(tblB.at[gBp_v], rBp, sem)
        cpA.wait()
        cpBn.wait()
        cpBp.wait()

        def row_body(i, a):
            iota = lax.iota(jnp.int32, 16)
            rvec = i * 16 + iota
            t0 = i * 16
            u0 = t0 // NB_SAMPLE
            ur = t0 % NB_SAMPLE
            uvec = (u0
                    + jnp.where(ur + iota >= NB_SAMPLE, 1, 0)
                    + jnp.where(ur + iota >= 2 * NB_SAMPLE, 1, 0))
            wA = wA_v[pl.ds(t0, 16)]
            wBn = wBn_v[pl.ds(t0, 16)]
            wBp = plsc.load_gather(wBp_v, [uvec])
            sacc = jnp.zeros((16,), jnp.float32)
            for dd in range(D):
                dv = (iota + dd) & 15
                aa = plsc.load_gather(rA, [rvec, wA + dv])
                bn = plsc.load_gather(rBn, [rvec, wBn + dv])
                bp = plsc.load_gather(rBp, [uvec, wBp + dv])
                sacc = sacc + (bn - bp) * (aa + aa - bp - bn)
            return a + jnp.maximum(sacc + ALPHA, 0.0)

        return lax.fori_loop(0, C // 16, row_body, acc)

    acc = lax.fori_loop(0, CPW, chunk_body, jnp.zeros((16,), jnp.float32))
    accv[...] = acc
    pltpu.sync_copy(accv, out_h.at[wid])


def kernel(outA, outB, matchA, matchB, nonMatchA, nonMatchB, device):
    tblA = outA.reshape(G, 128)
    tblB = outB.reshape(G, 128)
    idxA = jnp.pad(nonMatchA.reshape(-1), (0, PAD_TOTAL - TOTAL))
    idxBn = jnp.pad(nonMatchB.reshape(-1), (0, PAD_TOTAL - TOTAL))
    mB = jnp.pad(matchB.reshape(-1), (0, (PAD_TOTAL - TOTAL) // NB_SAMPLE))
    partials = _triplet_sc(tblA, tblB, idxA, mB, idxBn)
    pad_fix = 0.5 * (PAD_TOTAL - TOTAL)
    return (jnp.sum(partials) - pad_fix) / NB_MATCH
